# baseline (speedup 1.0000x reference)
import functools
import jax
import jax.numpy as jnp
from jax import lax
from jax.experimental import pallas as pl
from jax.experimental.pallas import tpu as pltpu
from jax.experimental.pallas import tpu_sc as plsc

_B, _L, _V, _D = 1024, 200, 6, 128
_LANES = 16
_NCHUNK = 13
_NW = 32
_SCN = 128
_BPW = _SCN // _NW
_TCB = 64


def _sc_body(seq_hbm, table_hbm, out_hbm, seqbuf, tbuf, trep, obuf0, obuf1, sem0, sem1):
    c = lax.axis_index("c")
    s = lax.axis_index("s")
    wid = s * 2 + c
    base = wid * _BPW

    pltpu.sync_copy(table_hbm, tbuf)
    pltpu.sync_copy(seq_hbm.at[pl.ds(base, _BPW)], seqbuf)

    lane = lax.iota(jnp.int32, _LANES)
    for v in range(_V):
        for cch in range(_D // _LANES):
            val = tbuf[v, pl.ds(cch * _LANES, _LANES)]
            addr = (lax.iota(jnp.int32, _LANES) + (v * _D + cch * _LANES)) * _LANES
            for j in range(_LANES):
                plsc.store_scatter(trep, [addr + j], val)

    starts = [min(ci * _LANES, _L - _LANES) for ci in range(_NCHUNK)]
    bufs = (obuf0, obuf1)
    sems = (sem0, sem1)

    def compute_tile(i, obuf):
        idxs = [seqbuf[i, pl.ds(st, _LANES)] * (_D * _LANES) for st in starts]

        @plsc.parallel_loop(0, _D // 16)
        def d_body(d0):
            dbase = d0 * 16
            for dd in range(16):
                dvec = lane + (dbase + dd) * _LANES
                for ci, st in enumerate(starts):
                    g = plsc.load_gather(trep, [idxs[ci] + dvec])
                    obuf[dbase + dd, pl.ds(st, _LANES)] = g

    def batch_pair(t, carry):
        for k in range(2):
            i = t * 2 + k
            buf, sem = bufs[k], sems[k]

            @pl.when(t > 0)
            def _wait_prev():
                pltpu.make_async_copy(buf, out_hbm.at[base + i - 2], sem).wait()

            compute_tile(i, buf)
            pltpu.async_copy(buf, out_hbm.at[base + i], sem)
        return carry

    lax.fori_loop(0, _BPW // 2, batch_pair, 0)
    pltpu.make_async_copy(obuf0, out_hbm.at[base + _BPW - 2], sem0).wait()
    pltpu.make_async_copy(obuf1, out_hbm.at[base + _BPW - 1], sem1).wait()


def _sc_lookup(seq, table):
    mesh = plsc.VectorSubcoreMesh(core_axis_name="c", subcore_axis_name="s")
    run = functools.partial(
        pl.kernel,
        mesh=mesh,
        compiler_params=pltpu.CompilerParams(needs_layout_passes=False),
        out_type=jax.ShapeDtypeStruct((_B, _D, _L), jnp.float32),
        scratch_types=[
            pltpu.VMEM((_BPW, _L), jnp.int32),
            pltpu.VMEM((_V, _D), jnp.float32),
            pltpu.VMEM((_V * _D * _LANES,), jnp.float32),
            pltpu.VMEM((_D, _L), jnp.float32),
            pltpu.VMEM((_D, _L), jnp.float32),
            pltpu.SemaphoreType.DMA,
            pltpu.SemaphoreType.DMA,
        ],
    )(_sc_body)
    return run(seq, table)


def _tc_body(seq_ref, table_ref, scout_ref, out_ref):
    del scout_ref
    m = seq_ref[...][:, None, :]
    acc = jnp.broadcast_to(table_ref[0, :][None, :, None], (_TCB, _D, _L))
    for v in range(1, _V):
        tv = table_ref[v, :][None, :, None]
        acc = jnp.where(m == v, tv, acc)
    out_ref[...] = acc


def _tc_fill(seq, table, sc_out):
    nblk = (_B - _SCN) // _TCB
    off = _SCN // _TCB
    return pl.pallas_call(
        _tc_body,
        grid=(nblk,),
        in_specs=[
            pl.BlockSpec((_TCB, _L), lambda g: (off + g, 0)),
            pl.BlockSpec((_V, _D), lambda g: (0, 0)),
            pl.BlockSpec(memory_space=pl.ANY),
        ],
        out_specs=pl.BlockSpec((_TCB, _D, _L), lambda g: (off + g, 0, 0)),
        out_shape=jax.ShapeDtypeStruct((_B, _D, _L), jnp.float32),
        input_output_aliases={2: 0},
    )(seq, table, sc_out)


def kernel(seq, table):
    seq = seq.astype(jnp.int32)
    sc_out = _sc_lookup(seq, table)
    return _tc_fill(seq, table, sc_out)


# speedup vs baseline: 1.0161x; 1.0161x over previous
import functools
import jax
import jax.numpy as jnp
from jax import lax
from jax.experimental import pallas as pl
from jax.experimental.pallas import tpu as pltpu
from jax.experimental.pallas import tpu_sc as plsc

_B, _L, _V, _D = 1024, 200, 6, 128
_LANES = 16
_NCHUNK = 13
_NW = 32
_SCN = 128
_BPW = _SCN // _NW
_TCB = 32


def _sc_body(seq_hbm, table_hbm, out_hbm, seqbuf, tbuf, trep, obuf0, obuf1, sem0, sem1):
    c = lax.axis_index("c")
    s = lax.axis_index("s")
    wid = s * 2 + c
    base = wid * _BPW

    pltpu.sync_copy(table_hbm, tbuf)
    pltpu.sync_copy(seq_hbm.at[pl.ds(base, _BPW)], seqbuf)

    lane = lax.iota(jnp.int32, _LANES)
    for v in range(_V):
        for cch in range(_D // _LANES):
            val = tbuf[v, pl.ds(cch * _LANES, _LANES)]
            addr = (lax.iota(jnp.int32, _LANES) + (v * _D + cch * _LANES)) * _LANES
            for j in range(_LANES):
                plsc.store_scatter(trep, [addr + j], val)

    starts = [min(ci * _LANES, _L - _LANES) for ci in range(_NCHUNK)]
    bufs = (obuf0, obuf1)
    sems = (sem0, sem1)

    def compute_tile(i, obuf):
        idxs = [seqbuf[i, pl.ds(st, _LANES)] * (_D * _LANES) for st in starts]

        @plsc.parallel_loop(0, _D // 16)
        def d_body(d0):
            dbase = d0 * 16
            for dd in range(16):
                dvec = lane + (dbase + dd) * _LANES
                for ci, st in enumerate(starts):
                    g = plsc.load_gather(trep, [idxs[ci] + dvec])
                    obuf[dbase + dd, pl.ds(st, _LANES)] = g

    def batch_pair(t, carry):
        for k in range(2):
            i = t * 2 + k
            buf, sem = bufs[k], sems[k]

            @pl.when(t > 0)
            def _wait_prev():
                pltpu.make_async_copy(buf, out_hbm.at[base + i - 2], sem).wait()

            compute_tile(i, buf)
            pltpu.async_copy(buf, out_hbm.at[base + i], sem)
        return carry

    lax.fori_loop(0, _BPW // 2, batch_pair, 0)
    pltpu.make_async_copy(obuf0, out_hbm.at[base + _BPW - 2], sem0).wait()
    pltpu.make_async_copy(obuf1, out_hbm.at[base + _BPW - 1], sem1).wait()


def _sc_lookup(seq, table):
    mesh = plsc.VectorSubcoreMesh(core_axis_name="c", subcore_axis_name="s")
    run = functools.partial(
        pl.kernel,
        mesh=mesh,
        compiler_params=pltpu.CompilerParams(needs_layout_passes=False),
        out_type=jax.ShapeDtypeStruct((_B, _D, _L), jnp.float32),
        scratch_types=[
            pltpu.VMEM((_BPW, _L), jnp.int32),
            pltpu.VMEM((_V, _D), jnp.float32),
            pltpu.VMEM((_V * _D * _LANES,), jnp.float32),
            pltpu.VMEM((_D, _L), jnp.float32),
            pltpu.VMEM((_D, _L), jnp.float32),
            pltpu.SemaphoreType.DMA,
            pltpu.SemaphoreType.DMA,
        ],
    )(_sc_body)
    return run(seq, table)


def _tc_body(seq_ref, table_ref, scout_ref, out_ref):
    del scout_ref
    m = seq_ref[...][:, None, :]
    acc = jnp.broadcast_to(table_ref[0, :][None, :, None], (_TCB, _D, _L))
    for v in range(1, _V):
        tv = table_ref[v, :][None, :, None]
        acc = jnp.where(m == v, tv, acc)
    out_ref[...] = acc


def _tc_fill(seq, table, sc_out):
    nblk = (_B - _SCN) // _TCB
    off = _SCN // _TCB
    return pl.pallas_call(
        _tc_body,
        grid=(nblk,),
        in_specs=[
            pl.BlockSpec((_TCB, _L), lambda g: (off + g, 0)),
            pl.BlockSpec((_V, _D), lambda g: (0, 0)),
            pl.BlockSpec(memory_space=pl.ANY),
        ],
        out_specs=pl.BlockSpec((_TCB, _D, _L), lambda g: (off + g, 0, 0)),
        out_shape=jax.ShapeDtypeStruct((_B, _D, _L), jnp.float32),
        input_output_aliases={2: 0},
    )(seq, table, sc_out)


def kernel(seq, table):
    seq = seq.astype(jnp.int32)
    sc_out = _sc_lookup(seq, table)
    return _tc_fill(seq, table, sc_out)


# speedup vs baseline: 1.0376x; 1.0212x over previous
import functools
import jax
import jax.numpy as jnp
from jax import lax
from jax.experimental import pallas as pl
from jax.experimental.pallas import tpu as pltpu
from jax.experimental.pallas import tpu_sc as plsc

_B, _L, _V, _D = 1024, 200, 6, 128
_LANES = 16
_NCHUNK = 13
_NW = 32
_SCN = 64
_BPW = _SCN // _NW
_TCB = 32


def _sc_body(seq_hbm, table_hbm, out_hbm, seqbuf, tbuf, trep, obuf0, obuf1, sem0, sem1):
    c = lax.axis_index("c")
    s = lax.axis_index("s")
    wid = s * 2 + c
    base = wid * _BPW

    pltpu.sync_copy(table_hbm, tbuf)
    pltpu.sync_copy(seq_hbm.at[pl.ds(base, _BPW)], seqbuf)

    lane = lax.iota(jnp.int32, _LANES)
    for v in range(_V):
        for cch in range(_D // _LANES):
            val = tbuf[v, pl.ds(cch * _LANES, _LANES)]
            addr = (lax.iota(jnp.int32, _LANES) + (v * _D + cch * _LANES)) * _LANES
            for j in range(_LANES):
                plsc.store_scatter(trep, [addr + j], val)

    starts = [min(ci * _LANES, _L - _LANES) for ci in range(_NCHUNK)]
    bufs = (obuf0, obuf1)
    sems = (sem0, sem1)

    def compute_tile(i, obuf):
        idxs = [seqbuf[i, pl.ds(st, _LANES)] * (_D * _LANES) for st in starts]

        @plsc.parallel_loop(0, _D // 16)
        def d_body(d0):
            dbase = d0 * 16
            for dd in range(16):
                dvec = lane + (dbase + dd) * _LANES
                for ci, st in enumerate(starts):
                    g = plsc.load_gather(trep, [idxs[ci] + dvec])
                    obuf[dbase + dd, pl.ds(st, _LANES)] = g

    def batch_pair(t, carry):
        for k in range(2):
            i = t * 2 + k
            buf, sem = bufs[k], sems[k]

            @pl.when(t > 0)
            def _wait_prev():
                pltpu.make_async_copy(buf, out_hbm.at[base + i - 2], sem).wait()

            compute_tile(i, buf)
            pltpu.async_copy(buf, out_hbm.at[base + i], sem)
        return carry

    lax.fori_loop(0, _BPW // 2, batch_pair, 0)
    pltpu.make_async_copy(obuf0, out_hbm.at[base + _BPW - 2], sem0).wait()
    pltpu.make_async_copy(obuf1, out_hbm.at[base + _BPW - 1], sem1).wait()


def _sc_lookup(seq, table):
    mesh = plsc.VectorSubcoreMesh(core_axis_name="c", subcore_axis_name="s")
    run = functools.partial(
        pl.kernel,
        mesh=mesh,
        compiler_params=pltpu.CompilerParams(needs_layout_passes=False),
        out_type=jax.ShapeDtypeStruct((_B, _D, _L), jnp.float32),
        scratch_types=[
            pltpu.VMEM((_BPW, _L), jnp.int32),
            pltpu.VMEM((_V, _D), jnp.float32),
            pltpu.VMEM((_V * _D * _LANES,), jnp.float32),
            pltpu.VMEM((_D, _L), jnp.float32),
            pltpu.VMEM((_D, _L), jnp.float32),
            pltpu.SemaphoreType.DMA,
            pltpu.SemaphoreType.DMA,
        ],
    )(_sc_body)
    return run(seq, table)


def _tc_body(seq_ref, table_ref, scout_ref, out_ref):
    del scout_ref
    m = seq_ref[...][:, None, :]
    acc = jnp.broadcast_to(table_ref[0, :][None, :, None], (_TCB, _D, _L))
    for v in range(1, _V):
        tv = table_ref[v, :][None, :, None]
        acc = jnp.where(m == v, tv, acc)
    out_ref[...] = acc


def _tc_fill(seq, table, sc_out):
    nblk = (_B - _SCN) // _TCB
    off = _SCN // _TCB
    return pl.pallas_call(
        _tc_body,
        grid=(nblk,),
        in_specs=[
            pl.BlockSpec((_TCB, _L), lambda g: (off + g, 0)),
            pl.BlockSpec((_V, _D), lambda g: (0, 0)),
            pl.BlockSpec(memory_space=pl.ANY),
        ],
        out_specs=pl.BlockSpec((_TCB, _D, _L), lambda g: (off + g, 0, 0)),
        out_shape=jax.ShapeDtypeStruct((_B, _D, _L), jnp.float32),
        input_output_aliases={2: 0},
    )(seq, table, sc_out)


def kernel(seq, table):
    seq = seq.astype(jnp.int32)
    sc_out = _sc_lookup(seq, table)
    return _tc_fill(seq, table, sc_out)
